# Initial kernel scaffold; baseline (speedup 1.0000x reference)
import jax
import jax.numpy as jnp
from jax.experimental import pallas as pl


def kernel(feature_s2t, feature_target, seg_s2t, seg_target, centroid_convert, centroid_target):
    raise NotImplementedError("write your pallas kernel here")


# speedup vs baseline: 1.3501x; 1.3501x over previous
import functools

import jax
import jax.numpy as jnp
from jax.experimental import pallas as pl


def _assign_upsample_kernel(f_ref, cent_ref, out_ref, *, bh, w, k, fac):
    cent = cent_ref[...]
    c2 = jnp.sum(cent * cent, axis=1, keepdims=True)
    kid = jax.lax.broadcasted_iota(jnp.int32, (k, w), 0)
    rows = []
    for r in range(bh):
        fr = f_ref[0, :, r, :]
        cross = jnp.dot(cent, fr)
        f2 = jnp.sum(fr * fr, axis=0, keepdims=True)
        d2 = jnp.maximum(f2 + c2 - 2.0 * cross, 1e-12)
        dmin = jnp.min(d2, axis=0, keepdims=True)
        m = jnp.min(jnp.where(d2 == dmin, kid, k), axis=0, keepdims=True)
        rows.append(m)
    mask = jnp.concatenate(rows, axis=0).astype(jnp.float32)
    col = jax.lax.broadcasted_iota(jnp.int32, (w, w * fac), 1)
    row = jax.lax.broadcasted_iota(jnp.int32, (w, w * fac), 0)
    sel = (col // fac == row).astype(jnp.float32)
    rep = jnp.dot(mask, sel,
                  precision=jax.lax.Precision.HIGHEST)
    rep = rep.astype(jnp.int32)
    rep3 = jnp.broadcast_to(rep[:, None, :], (bh, fac, w * fac))
    out_ref[0] = rep3.reshape(bh * fac, w * fac)


def _assign_and_upsample(feature, centroid, H, W):
    b, c, h, w = feature.shape
    k = centroid.shape[0]
    fac = H // h
    assert H == h * fac and W == w * (W // w) and W // w == fac
    bh = 8
    grid = (b, h // bh)
    return pl.pallas_call(
        functools.partial(_assign_upsample_kernel, bh=bh, w=w, k=k, fac=fac),
        grid=grid,
        in_specs=[
            pl.BlockSpec((1, c, bh, w), lambda i, j: (i, 0, j, 0)),
            pl.BlockSpec((k, c), lambda i, j: (0, 0)),
        ],
        out_specs=pl.BlockSpec((1, bh * fac, w * fac), lambda i, j: (i, j, 0)),
        out_shape=jax.ShapeDtypeStruct((b, H, W), jnp.int32),
    )(feature, centroid)


def kernel(feature_s2t, feature_target, seg_s2t, seg_target, centroid_convert, centroid_target):
    H1, W1 = seg_s2t.shape[1], seg_s2t.shape[2]
    H2, W2 = seg_target.shape[1], seg_target.shape[2]
    mask_s2t_target = _assign_and_upsample(feature_s2t, centroid_target, H1, W1)
    mask_target_s2t = _assign_and_upsample(feature_target, centroid_convert, H2, W2)
    return (mask_s2t_target, mask_target_s2t)


# speedup vs baseline: 1.3814x; 1.0232x over previous
import functools

import jax
import jax.numpy as jnp
from jax.experimental import pallas as pl


def _assign_upsample_kernel(f_ref, cent_ref, out_ref, *, bh, w, k, fac):
    cent = cent_ref[...]
    c2 = jnp.sum(cent * cent, axis=1, keepdims=True)
    f = f_ref[0]
    cross = jnp.dot(cent, f)
    score = c2 - 2.0 * cross
    smin = jnp.min(score, axis=0, keepdims=True)
    kid = jax.lax.broadcasted_iota(jnp.int32, (k, bh * w), 0)
    m = jnp.min(jnp.where(score == smin, kid, k), axis=0, keepdims=True)
    mf = m.astype(jnp.float32)
    rows = [mf[:, r * w:(r + 1) * w] for r in range(bh)]
    mask = jnp.concatenate(rows, axis=0)
    col = jax.lax.broadcasted_iota(jnp.int32, (w, w * fac), 1)
    row = jax.lax.broadcasted_iota(jnp.int32, (w, w * fac), 0)
    sel = (col // fac == row).astype(jnp.float32)
    rep = jnp.dot(mask, sel).astype(jnp.int32)
    rep3 = jnp.broadcast_to(rep[:, None, :], (bh, fac, w * fac))
    out_ref[0] = rep3.reshape(bh * fac, w * fac)


def _assign_and_upsample(feature, centroid, H, W):
    b, c, h, w = feature.shape
    k = centroid.shape[0]
    fac = H // h
    assert H == h * fac and W == w * fac
    bh = 16
    fflat = feature.reshape(b, c, h * w)
    return pl.pallas_call(
        functools.partial(_assign_upsample_kernel, bh=bh, w=w, k=k, fac=fac),
        grid=(b, h // bh),
        in_specs=[
            pl.BlockSpec((1, c, bh * w), lambda i, j: (i, 0, j)),
            pl.BlockSpec((k, c), lambda i, j: (0, 0)),
        ],
        out_specs=pl.BlockSpec((1, bh * fac, w * fac), lambda i, j: (i, j, 0)),
        out_shape=jax.ShapeDtypeStruct((b, H, W), jnp.int32),
    )(fflat, centroid)


def kernel(feature_s2t, feature_target, seg_s2t, seg_target, centroid_convert, centroid_target):
    H1, W1 = seg_s2t.shape[1], seg_s2t.shape[2]
    H2, W2 = seg_target.shape[1], seg_target.shape[2]
    mask_s2t_target = _assign_and_upsample(feature_s2t, centroid_target, H1, W1)
    mask_target_s2t = _assign_and_upsample(feature_target, centroid_convert, H2, W2)
    return (mask_s2t_target, mask_target_s2t)
